# Initial kernel scaffold; baseline (speedup 1.0000x reference)
import jax
import jax.numpy as jnp
from jax.experimental import pallas as pl


def kernel(lex_indices, x, W):
    raise NotImplementedError("write your pallas kernel here")


# speedup vs baseline: 3.1286x; 3.1286x over previous
import functools

import jax
import jax.numpy as jnp
from jax import lax
from jax.experimental import pallas as pl
from jax.experimental.pallas import tpu as pltpu
from jax.experimental.pallas import tpu_sc as plsc

_EPS = 1e-7
_LANES = 16


def _sc_scores_kernel(idx_hbm, w_hbm, out_hbm, idx_v, sc_v, wtab_v, wsum_v, *,
                      n_workers, n_cores, samples_per_worker, seq_len, vocab,
                      nlex):
    n_vec = (seq_len + _LANES - 1) // _LANES
    rem = seq_len - (n_vec - 1) * _LANES
    chunk = samples_per_worker * seq_len

    wid = lax.axis_index("s") * n_cores + lax.axis_index("c")
    base = wid * chunk

    pltpu.sync_copy(idx_hbm.at[pl.ds(base, chunk)], idx_v.at[pl.ds(0, chunk)])
    pltpu.sync_copy(w_hbm, wtab_v)
    idx_v[pl.ds(chunk, _LANES)] = jnp.zeros((_LANES,), jnp.int32)

    lanes = lax.broadcasted_iota(jnp.int32, (_LANES,), 0)

    for i in range(vocab // _LANES):
        v = lanes + (i * _LANES)
        flat = v * nlex
        acc = plsc.load_gather(wtab_v, [flat])
        for n in range(1, nlex):
            acc = acc + plsc.load_gather(wtab_v, [flat + n])
        wsum_v[pl.ds(i * _LANES, _LANES)] = acc

    last_mask = lanes < rem

    def body(sidx, carry):
        tbase = sidx * seq_len
        acc = jnp.zeros((_LANES,), jnp.float32)
        evs = []
        for j in range(n_vec):
            iv = idx_v[pl.ds(tbase + j * _LANES, _LANES)]
            s = plsc.load_gather(wsum_v, [iv])
            e2 = jnp.exp(-2.0 * jnp.abs(s))
            th = (1.0 - e2) / (1.0 + e2)
            th = jnp.where(s < 0.0, -th, th)
            ev = jnp.exp(th)
            if j == n_vec - 1:
                ev = jnp.where(last_mask, ev, 0.0)
            evs.append(ev)
            acc = acc + ev
        inv = 1.0 / jnp.broadcast_to(jnp.sum(acc) + _EPS, (_LANES,))
        for j in range(n_vec):
            sc_v[pl.ds(tbase + j * _LANES, _LANES)] = evs[j] * inv
        return carry

    lax.fori_loop(0, samples_per_worker, body, 0)
    pltpu.sync_copy(sc_v.at[pl.ds(0, chunk)], out_hbm.at[pl.ds(base, chunk)])


def _tc_pool_kernel(a_ref, x_ref, o_ref):
    o_ref[...] = jnp.sum(x_ref[...] * a_ref[...], axis=1)


def kernel(lex_indices, x, W):
    B, L = lex_indices.shape
    _, _, D = x.shape
    V, NLEX = W.shape

    info = plsc.get_sparse_core_info()
    nc, ns = info.num_cores, info.num_subcores
    nw = nc * ns
    spw = B // nw
    chunk = spw * L
    pad_chunk = chunk + _LANES

    mesh = plsc.VectorSubcoreMesh(core_axis_name="c", subcore_axis_name="s")
    sc_scores = pl.kernel(
        functools.partial(
            _sc_scores_kernel,
            n_workers=nw, n_cores=nc, samples_per_worker=spw,
            seq_len=L, vocab=V, nlex=NLEX),
        out_type=jax.ShapeDtypeStruct((B * L,), jnp.float32),
        mesh=mesh,
        compiler_params=pltpu.CompilerParams(needs_layout_passes=False),
        scratch_types=[
            pltpu.VMEM((pad_chunk,), jnp.int32),
            pltpu.VMEM((pad_chunk,), jnp.float32),
            pltpu.VMEM((V * NLEX,), jnp.float32),
            pltpu.VMEM((V,), jnp.float32),
        ],
    )
    a = sc_scores(lex_indices.reshape(B * L), W.reshape(V * NLEX))
    a3 = a.reshape(B, L, 1)

    BLK = 128
    out = pl.pallas_call(
        _tc_pool_kernel,
        grid=(B // BLK,),
        in_specs=[
            pl.BlockSpec((BLK, L, 1), lambda i: (i, 0, 0)),
            pl.BlockSpec((BLK, L, D), lambda i: (i, 0, 0)),
        ],
        out_specs=pl.BlockSpec((BLK, D), lambda i: (i, 0)),
        out_shape=jax.ShapeDtypeStruct((B, D), jnp.float32),
    )(a3, x)
    return out


# speedup vs baseline: 5.5568x; 1.7761x over previous
import functools

import jax
import jax.numpy as jnp
from jax import lax
from jax.experimental import pallas as pl
from jax.experimental.pallas import tpu as pltpu
from jax.experimental.pallas import tpu_sc as plsc

_EPS = 1e-7
_LANES = 16


def _sc_scores_kernel(idx_hbm, w_hbm, out_hbm, idx_v, sc_v, wtab_v, wsum_v, *,
                      n_workers, n_cores, samples_per_worker, seq_len, vocab,
                      nlex):
    n_vec = (seq_len + _LANES - 1) // _LANES
    rem = seq_len - (n_vec - 1) * _LANES
    chunk = samples_per_worker * seq_len

    wid = lax.axis_index("s") * n_cores + lax.axis_index("c")
    base = wid * chunk

    pltpu.sync_copy(idx_hbm.at[pl.ds(base, chunk)], idx_v.at[pl.ds(0, chunk)])
    pltpu.sync_copy(w_hbm, wtab_v)
    idx_v[pl.ds(chunk, _LANES)] = jnp.zeros((_LANES,), jnp.int32)

    lanes = lax.broadcasted_iota(jnp.int32, (_LANES,), 0)

    for i in range(vocab // _LANES):
        v = lanes + (i * _LANES)
        flat = v * nlex
        acc = plsc.load_gather(wtab_v, [flat])
        for n in range(1, nlex):
            acc = acc + plsc.load_gather(wtab_v, [flat + n])
        wsum_v[pl.ds(i * _LANES, _LANES)] = acc

    last_mask = lanes < rem

    def body(sidx, carry):
        tbase = sidx * seq_len
        acc = jnp.zeros((_LANES,), jnp.float32)
        evs = []
        for j in range(n_vec):
            iv = idx_v[pl.ds(tbase + j * _LANES, _LANES)]
            s = plsc.load_gather(wsum_v, [iv])
            e2 = jnp.exp(-2.0 * jnp.abs(s))
            th = (1.0 - e2) / (1.0 + e2)
            th = jnp.where(s < 0.0, -th, th)
            ev = jnp.exp(th)
            if j == n_vec - 1:
                ev = jnp.where(last_mask, ev, 0.0)
            evs.append(ev)
            acc = acc + ev
        inv = 1.0 / jnp.broadcast_to(jnp.sum(acc) + _EPS, (_LANES,))
        for j in range(n_vec):
            sc_v[pl.ds(tbase + j * _LANES, _LANES)] = evs[j] * inv
        return carry

    lax.fori_loop(0, samples_per_worker, body, 0)
    pltpu.sync_copy(sc_v.at[pl.ds(0, chunk)], out_hbm.at[pl.ds(base, chunk)])


def _tc_pool_kernel(a_ref, x_ref, o_ref, *, blk, seq_len, d):
    m = blk * seq_len
    af = a_ref[...].reshape(1, m)
    rows = lax.broadcasted_iota(jnp.int32, (blk, m), 0)
    cols = lax.broadcasted_iota(jnp.int32, (blk, m), 1)
    lo = rows * seq_len
    mask = (cols >= lo) & (cols < lo + seq_len)
    w = jnp.where(mask, jnp.broadcast_to(af, (blk, m)), 0.0)
    xv = x_ref[...].reshape(m, d)
    o_ref[...] = jnp.dot(w, xv, preferred_element_type=jnp.float32)


def kernel(lex_indices, x, W):
    B, L = lex_indices.shape
    _, _, D = x.shape
    V, NLEX = W.shape

    info = plsc.get_sparse_core_info()
    nc, ns = info.num_cores, info.num_subcores
    nw = nc * ns
    spw = B // nw
    chunk = spw * L
    pad_chunk = chunk + _LANES

    mesh = plsc.VectorSubcoreMesh(core_axis_name="c", subcore_axis_name="s")
    sc_scores = pl.kernel(
        functools.partial(
            _sc_scores_kernel,
            n_workers=nw, n_cores=nc, samples_per_worker=spw,
            seq_len=L, vocab=V, nlex=NLEX),
        out_type=jax.ShapeDtypeStruct((B * L,), jnp.float32),
        mesh=mesh,
        compiler_params=pltpu.CompilerParams(needs_layout_passes=False),
        scratch_types=[
            pltpu.VMEM((pad_chunk,), jnp.int32),
            pltpu.VMEM((pad_chunk,), jnp.float32),
            pltpu.VMEM((V * NLEX,), jnp.float32),
            pltpu.VMEM((V,), jnp.float32),
        ],
    )
    a = sc_scores(lex_indices.reshape(B * L), W.reshape(V * NLEX))

    BLK = 16
    a2 = a.reshape(B // BLK, 1, BLK * L)
    out = pl.pallas_call(
        functools.partial(_tc_pool_kernel, blk=BLK, seq_len=L, d=D),
        grid=(B // BLK,),
        in_specs=[
            pl.BlockSpec((1, 1, BLK * L), lambda i: (i, 0, 0)),
            pl.BlockSpec((BLK, L, D), lambda i: (i, 0, 0)),
        ],
        out_specs=pl.BlockSpec((BLK, D), lambda i: (i, 0)),
        out_shape=jax.ShapeDtypeStruct((B, D), jnp.float32),
    )(a2, x)
    return out


# speedup vs baseline: 24.1776x; 4.3510x over previous
import functools

import jax
import jax.numpy as jnp
from jax import lax
from jax.experimental import pallas as pl
from jax.experimental.pallas import tpu as pltpu
from jax.experimental.pallas import tpu_sc as plsc

_EPS = 1e-7
_LANES = 16


def _sc_scores_kernel(idx_hbm, w_hbm, out_hbm, idx_v, sc_v, wtab_v, etab_v, *,
                      n_cores, samples_per_worker, seq_len, vocab, nlex):
    chunk = samples_per_worker * seq_len
    n_groups = samples_per_worker // _LANES

    wid = lax.axis_index("s") * n_cores + lax.axis_index("c")
    base = wid * chunk

    pltpu.sync_copy(idx_hbm.at[pl.ds(base, chunk)], idx_v)
    pltpu.sync_copy(w_hbm, wtab_v)

    lanes = lax.broadcasted_iota(jnp.int32, (_LANES,), 0)

    for i in range(vocab // _LANES):
        flat = (lanes + i * _LANES) * nlex
        s = plsc.load_gather(wtab_v, [flat])
        for n in range(1, nlex):
            s = s + plsc.load_gather(wtab_v, [flat + n])
        e2 = jnp.exp(-2.0 * jnp.abs(s))
        th = (1.0 - e2) / (1.0 + e2)
        th = jnp.where(s < 0.0, -th, th)
        etab_v[pl.ds(i * _LANES, _LANES)] = jnp.exp(th)

    for g in range(n_groups):
        sbase = (lanes + g * _LANES) * seq_len

        def body(l, acc, sbase=sbase, g=g):
            iv = plsc.load_gather(idx_v, [sbase + l])
            ev = plsc.load_gather(etab_v, [iv])
            sc_v[pl.ds(l * samples_per_worker + g * _LANES, _LANES)] = ev
            return acc + ev

        acc = lax.fori_loop(0, seq_len, body, jnp.zeros((_LANES,), jnp.float32))
        inv = 1.0 / (acc + _EPS)

        def scale(l, c, inv=inv, g=g):
            off = l * samples_per_worker + g * _LANES
            sc_v[pl.ds(off, _LANES)] = sc_v[pl.ds(off, _LANES)] * inv
            return c

        lax.fori_loop(0, seq_len, scale, 0)

    pltpu.sync_copy(sc_v, out_hbm.at[pl.ds(base, chunk)])


def _tc_pool_kernel(a_ref, x_ref, o_ref):
    av = a_ref[0]
    xv = x_ref[...]
    o_ref[...] = jnp.sum(xv * jnp.broadcast_to(av, xv.shape), axis=0)


def kernel(lex_indices, x, W):
    B, L = lex_indices.shape
    _, _, D = x.shape
    V, NLEX = W.shape

    info = plsc.get_sparse_core_info()
    nc, ns = info.num_cores, info.num_subcores
    nw = nc * ns
    spw = B // nw
    chunk = spw * L

    mesh = plsc.VectorSubcoreMesh(core_axis_name="c", subcore_axis_name="s")
    sc_scores = pl.kernel(
        functools.partial(
            _sc_scores_kernel,
            n_cores=nc, samples_per_worker=spw,
            seq_len=L, vocab=V, nlex=NLEX),
        out_type=jax.ShapeDtypeStruct((B * L,), jnp.float32),
        mesh=mesh,
        compiler_params=pltpu.CompilerParams(needs_layout_passes=False),
        scratch_types=[
            pltpu.VMEM((chunk,), jnp.int32),
            pltpu.VMEM((chunk,), jnp.float32),
            pltpu.VMEM((V * NLEX,), jnp.float32),
            pltpu.VMEM((V,), jnp.float32),
        ],
    )
    a = sc_scores(lex_indices.reshape(B * L), W.reshape(V * NLEX))

    BW = spw
    a4 = a.reshape(nw, L, 1, BW)
    xt = x.transpose(1, 2, 0)
    out_t = pl.pallas_call(
        _tc_pool_kernel,
        grid=(nw,),
        in_specs=[
            pl.BlockSpec((1, L, 1, BW), lambda i: (i, 0, 0, 0)),
            pl.BlockSpec((L, D, BW), lambda i: (0, 0, i)),
        ],
        out_specs=pl.BlockSpec((D, BW), lambda i: (0, i)),
        out_shape=jax.ShapeDtypeStruct((D, B), jnp.float32),
    )(a4, xt)
    return out_t.T


# speedup vs baseline: 25.7524x; 1.0651x over previous
import functools

import jax
import jax.numpy as jnp
from jax import lax
from jax.experimental import pallas as pl
from jax.experimental.pallas import tpu as pltpu
from jax.experimental.pallas import tpu_sc as plsc

_EPS = 1e-7
_LANES = 16


def _sc_scores_kernel(idx_hbm, w_hbm, out_hbm, inv_hbm, idx_v, sc_v, wtab_v,
                      etab_v, inv_v, *,
                      n_cores, samples_per_worker, seq_len, vocab, nlex):
    chunk = samples_per_worker * seq_len
    n_groups = samples_per_worker // _LANES

    wid = lax.axis_index("s") * n_cores + lax.axis_index("c")
    base = wid * chunk

    pltpu.sync_copy(idx_hbm.at[pl.ds(base, chunk)], idx_v)
    pltpu.sync_copy(w_hbm, wtab_v)

    lanes = lax.broadcasted_iota(jnp.int32, (_LANES,), 0)

    for i in range(vocab // _LANES):
        flat = (lanes + i * _LANES) * nlex
        s = plsc.load_gather(wtab_v, [flat])
        for n in range(1, nlex):
            s = s + plsc.load_gather(wtab_v, [flat + n])
        e2 = jnp.exp(-2.0 * jnp.abs(s))
        th = (1.0 - e2) / (1.0 + e2)
        th = jnp.where(s < 0.0, -th, th)
        etab_v[pl.ds(i * _LANES, _LANES)] = jnp.exp(th)

    for g in range(n_groups):
        sbase = (lanes + g * _LANES) * seq_len

        def body(l, acc, sbase=sbase, g=g):
            iv = plsc.load_gather(idx_v, [sbase + l])
            ev = plsc.load_gather(etab_v, [iv])
            sc_v[pl.ds(l * samples_per_worker + g * _LANES, _LANES)] = ev
            return acc + ev

        acc = lax.fori_loop(0, seq_len, body, jnp.zeros((_LANES,), jnp.float32))
        inv_v[pl.ds(g * _LANES, _LANES)] = 1.0 / (acc + _EPS)

    pltpu.sync_copy(sc_v, out_hbm.at[pl.ds(base, chunk)])
    pltpu.sync_copy(inv_v, inv_hbm.at[pl.ds(wid * samples_per_worker,
                                            samples_per_worker)])


def _tc_pool_kernel(a_ref, i_ref, x_ref, o_ref):
    av = a_ref[0]
    xv = x_ref[...]
    s = jnp.sum(xv * jnp.broadcast_to(av, xv.shape), axis=0)
    o_ref[...] = s * jnp.broadcast_to(i_ref[0], s.shape)


def kernel(lex_indices, x, W):
    B, L = lex_indices.shape
    _, _, D = x.shape
    V, NLEX = W.shape

    info = plsc.get_sparse_core_info()
    nc, ns = info.num_cores, info.num_subcores
    nw = nc * ns
    spw = B // nw
    chunk = spw * L

    mesh = plsc.VectorSubcoreMesh(core_axis_name="c", subcore_axis_name="s")
    sc_scores = pl.kernel(
        functools.partial(
            _sc_scores_kernel,
            n_cores=nc, samples_per_worker=spw,
            seq_len=L, vocab=V, nlex=NLEX),
        out_type=(jax.ShapeDtypeStruct((B * L,), jnp.float32),
                  jax.ShapeDtypeStruct((B,), jnp.float32)),
        mesh=mesh,
        compiler_params=pltpu.CompilerParams(needs_layout_passes=False),
        scratch_types=[
            pltpu.VMEM((chunk,), jnp.int32),
            pltpu.VMEM((chunk,), jnp.float32),
            pltpu.VMEM((V * NLEX,), jnp.float32),
            pltpu.VMEM((V,), jnp.float32),
            pltpu.VMEM((spw,), jnp.float32),
        ],
    )
    a, inv = sc_scores(lex_indices.reshape(B * L), W.reshape(V * NLEX))

    BW = spw
    a4 = a.reshape(nw, L, 1, BW)
    inv3 = inv.reshape(nw, 1, BW)
    xt = x.transpose(1, 2, 0)
    out_t = pl.pallas_call(
        _tc_pool_kernel,
        grid=(nw,),
        in_specs=[
            pl.BlockSpec((1, L, 1, BW), lambda i: (i, 0, 0, 0)),
            pl.BlockSpec((1, 1, BW), lambda i: (i, 0, 0)),
            pl.BlockSpec((L, D, BW), lambda i: (0, 0, i)),
        ],
        out_specs=pl.BlockSpec((D, BW), lambda i: (0, i)),
        out_shape=jax.ShapeDtypeStruct((D, B), jnp.float32),
    )(a4, inv3, xt)
    return out_t.T


# speedup vs baseline: 25.9006x; 1.0058x over previous
import functools

import jax
import jax.numpy as jnp
from jax import lax
from jax.experimental import pallas as pl
from jax.experimental.pallas import tpu as pltpu
from jax.experimental.pallas import tpu_sc as plsc

_EPS = 1e-7
_LANES = 16


def _sc_scores_kernel(idx_hbm, w_hbm, out_hbm, inv_hbm, idx_v, sc_v, wtab_v,
                      etab_v, inv_v, *,
                      n_cores, samples_per_worker, seq_len, vocab, nlex):
    chunk = samples_per_worker * seq_len
    n_groups = samples_per_worker // _LANES

    wid = lax.axis_index("s") * n_cores + lax.axis_index("c")
    base = wid * chunk

    pltpu.sync_copy(idx_hbm.at[pl.ds(base, chunk)], idx_v)
    pltpu.sync_copy(w_hbm, wtab_v)

    lanes = lax.broadcasted_iota(jnp.int32, (_LANES,), 0)

    for i in range(vocab // _LANES):
        flat = (lanes + i * _LANES) * nlex
        s = plsc.load_gather(wtab_v, [flat])
        for n in range(1, nlex):
            s = s + plsc.load_gather(wtab_v, [flat + n])
        e2 = jnp.exp(-2.0 * jnp.abs(s))
        th = (1.0 - e2) / (1.0 + e2)
        th = jnp.where(s < 0.0, -th, th)
        etab_v[pl.ds(i * _LANES, _LANES)] = jnp.exp(th)

    unroll = 8
    for g in range(n_groups):
        sbase = (lanes + g * _LANES) * seq_len

        def body(j, acc, sbase=sbase, g=g):
            l0 = j * unroll
            for u in range(unroll):
                iv = plsc.load_gather(idx_v, [sbase + (l0 + u)])
                ev = plsc.load_gather(etab_v, [iv])
                sc_v[pl.ds((l0 + u) * samples_per_worker + g * _LANES,
                           _LANES)] = ev
                acc = acc + ev
            return acc

        acc = lax.fori_loop(0, seq_len // unroll, body,
                            jnp.zeros((_LANES,), jnp.float32))
        inv_v[pl.ds(g * _LANES, _LANES)] = 1.0 / (acc + _EPS)

    pltpu.sync_copy(sc_v, out_hbm.at[pl.ds(base, chunk)])
    pltpu.sync_copy(inv_v, inv_hbm.at[pl.ds(wid * samples_per_worker,
                                            samples_per_worker)])


def _tc_pool_kernel(a_ref, i_ref, x_ref, o_ref):
    av = a_ref[0]
    xv = x_ref[...]
    s = jnp.sum(xv * jnp.broadcast_to(av, xv.shape), axis=0)
    o_ref[...] = s * jnp.broadcast_to(i_ref[0], s.shape)


def kernel(lex_indices, x, W):
    B, L = lex_indices.shape
    _, _, D = x.shape
    V, NLEX = W.shape

    info = plsc.get_sparse_core_info()
    nc, ns = info.num_cores, info.num_subcores
    nw = nc * ns
    spw = B // nw
    chunk = spw * L

    mesh = plsc.VectorSubcoreMesh(core_axis_name="c", subcore_axis_name="s")
    sc_scores = pl.kernel(
        functools.partial(
            _sc_scores_kernel,
            n_cores=nc, samples_per_worker=spw,
            seq_len=L, vocab=V, nlex=NLEX),
        out_type=(jax.ShapeDtypeStruct((B * L,), jnp.float32),
                  jax.ShapeDtypeStruct((B,), jnp.float32)),
        mesh=mesh,
        compiler_params=pltpu.CompilerParams(needs_layout_passes=False),
        scratch_types=[
            pltpu.VMEM((chunk,), jnp.int32),
            pltpu.VMEM((chunk,), jnp.float32),
            pltpu.VMEM((V * NLEX,), jnp.float32),
            pltpu.VMEM((V,), jnp.float32),
            pltpu.VMEM((spw,), jnp.float32),
        ],
    )
    a, inv = sc_scores(lex_indices.reshape(B * L), W.reshape(V * NLEX))

    BW = spw
    a4 = a.reshape(nw, L, 1, BW)
    inv3 = inv.reshape(nw, 1, BW)
    xt = x.transpose(1, 2, 0)
    out_t = pl.pallas_call(
        _tc_pool_kernel,
        grid=(nw,),
        in_specs=[
            pl.BlockSpec((1, L, 1, BW), lambda i: (i, 0, 0, 0)),
            pl.BlockSpec((1, 1, BW), lambda i: (i, 0, 0)),
            pl.BlockSpec((L, D, BW), lambda i: (0, 0, i)),
        ],
        out_specs=pl.BlockSpec((D, BW), lambda i: (0, i)),
        out_shape=jax.ShapeDtypeStruct((D, B), jnp.float32),
    )(a4, inv3, xt)
    return out_t.T
